# baseline (speedup 1.0000x reference)
import jax
import jax.numpy as jnp
from jax.experimental import pallas as pl
from jax.experimental.pallas import tpu as pltpu

H, KVH, DH = 16, 8, 128
EPS = 1e-6
NEG = -1e30

BM_PROJ = 256
BM_Q = 512
BK = 512


def _rms_norm_heads(t, w, extra_scale):
    rows = t.shape[0]
    n = t.shape[1] // DH
    t3 = t.reshape(rows, n, DH)
    var = jnp.mean(t3 * t3, axis=-1, keepdims=True)
    t3 = t3 * (jax.lax.rsqrt(var + EPS) * extra_scale)
    return (t3 * w.reshape(1, 1, DH)).reshape(rows, n * DH)


def _qkv_proj_kernel(x_ref, wq_ref, wk_ref, wv_ref, qw_ref, kw_ref,
                     q_ref, k_ref, v_ref):
    xb = x_ref[...].astype(jnp.bfloat16)
    dims = (((1,), (1,)), ((), ()))
    q = jax.lax.dot_general(xb, wq_ref[...], dims,
                            preferred_element_type=jnp.float32)
    k = jax.lax.dot_general(xb, wk_ref[...], dims,
                            preferred_element_type=jnp.float32)
    v = jax.lax.dot_general(xb, wv_ref[...], dims,
                            preferred_element_type=jnp.float32)
    qn = _rms_norm_heads(q, qw_ref[...], DH ** -0.5)
    kn = _rms_norm_heads(k, kw_ref[...], 1.0)
    q_ref[...] = qn.astype(jnp.bfloat16)
    k_ref[...] = kn.astype(jnp.bfloat16)
    v_ref[...] = v.astype(jnp.bfloat16)


def _attn_kernel(q_ref, k_ref, v_ref, o_ref, acc_ref, l_ref):
    i = pl.program_id(1)
    j = pl.program_id(2)
    dims_nt = (((1,), (1,)), ((), ()))
    dims_nn = (((1,), (0,)), ((), ()))

    @pl.when(j == 0)
    def _init():
        acc_ref[...] = jnp.zeros_like(acc_ref)
        l_ref[...] = jnp.zeros_like(l_ref)

    @pl.when(j < i)
    def _full():
        s = jax.lax.dot_general(q_ref[...], k_ref[...], dims_nt,
                                preferred_element_type=jnp.float32)
        p = jnp.exp(s)
        l_ref[...] += jnp.sum(p, axis=-1, keepdims=True)
        acc_ref[...] += jax.lax.dot_general(
            p.astype(jnp.bfloat16), v_ref[...], dims_nn,
            preferred_element_type=jnp.float32)

    @pl.when(j == i)
    def _diag():
        s = jax.lax.dot_general(q_ref[...], k_ref[...], dims_nt,
                                preferred_element_type=jnp.float32)
        row = jax.lax.broadcasted_iota(jnp.int32, (BM_Q, BK), 0)
        col = jax.lax.broadcasted_iota(jnp.int32, (BM_Q, BK), 1)
        p = jnp.exp(jnp.where(row >= col, s, NEG))
        l = l_ref[...] + jnp.sum(p, axis=-1, keepdims=True)
        acc = acc_ref[...] + jax.lax.dot_general(
            p.astype(jnp.bfloat16), v_ref[...], dims_nn,
            preferred_element_type=jnp.float32)
        o_ref[...] = (acc / l).astype(jnp.bfloat16)


def _out_proj_kernel(a_ref, wo_ref, o_ref):
    o_ref[...] = jax.lax.dot_general(
        a_ref[...], wo_ref[...], (((1,), (1,)), ((), ())),
        preferred_element_type=jnp.float32)


def kernel(x, Wq, Wk, Wv, Wo, q_norm_w, k_norm_w):
    b, s, d = x.shape
    x2 = x.reshape(s, d)
    wq = Wq.astype(jnp.bfloat16)
    wk = Wk.astype(jnp.bfloat16)
    wv = Wv.astype(jnp.bfloat16)
    wo = Wo.astype(jnp.bfloat16)
    qw = q_norm_w.reshape(1, DH)
    kw = k_norm_w.reshape(1, DH)

    n_row_blocks = s // BM_PROJ
    q, k, v = pl.pallas_call(
        _qkv_proj_kernel,
        grid=(n_row_blocks,),
        in_specs=[
            pl.BlockSpec((BM_PROJ, d), lambda i: (i, 0)),
            pl.BlockSpec((H * DH, d), lambda i: (0, 0)),
            pl.BlockSpec((KVH * DH, d), lambda i: (0, 0)),
            pl.BlockSpec((KVH * DH, d), lambda i: (0, 0)),
            pl.BlockSpec((1, DH), lambda i: (0, 0)),
            pl.BlockSpec((1, DH), lambda i: (0, 0)),
        ],
        out_specs=[
            pl.BlockSpec((BM_PROJ, H * DH), lambda i: (i, 0)),
            pl.BlockSpec((BM_PROJ, KVH * DH), lambda i: (i, 0)),
            pl.BlockSpec((BM_PROJ, KVH * DH), lambda i: (i, 0)),
        ],
        out_shape=[
            jax.ShapeDtypeStruct((s, H * DH), jnp.bfloat16),
            jax.ShapeDtypeStruct((s, KVH * DH), jnp.bfloat16),
            jax.ShapeDtypeStruct((s, KVH * DH), jnp.bfloat16),
        ],
    )(x2, wq, wk, wv, qw, kw)

    n_q_blocks = s // BM_Q
    n_k_blocks = s // BK
    groups = H // KVH
    attn = pl.pallas_call(
        _attn_kernel,
        grid=(H, n_q_blocks, n_k_blocks),
        in_specs=[
            pl.BlockSpec((BM_Q, DH), lambda h, i, j: (i, h)),
            pl.BlockSpec((BK, DH), lambda h, i, j: (j, h // groups)),
            pl.BlockSpec((BK, DH), lambda h, i, j: (j, h // groups)),
        ],
        out_specs=pl.BlockSpec((BM_Q, DH), lambda h, i, j: (i, h)),
        out_shape=jax.ShapeDtypeStruct((s, H * DH), jnp.bfloat16),
        scratch_shapes=[
            pltpu.VMEM((BM_Q, DH), jnp.float32),
            pltpu.VMEM((BM_Q, 1), jnp.float32),
        ],
    )(q, k, v)

    out = pl.pallas_call(
        _out_proj_kernel,
        grid=(n_row_blocks,),
        in_specs=[
            pl.BlockSpec((BM_PROJ, H * DH), lambda i: (i, 0)),
            pl.BlockSpec((d, H * DH), lambda i: (0, 0)),
        ],
        out_specs=pl.BlockSpec((BM_PROJ, d), lambda i: (i, 0)),
        out_shape=jax.ShapeDtypeStruct((s, d), jnp.float32),
    )(attn, wo)

    return out.reshape(b, s, d)


# speedup vs baseline: 1.4697x; 1.4697x over previous
import jax
import jax.numpy as jnp
from jax.experimental import pallas as pl
from jax.experimental.pallas import tpu as pltpu

H, KVH, DH = 16, 8, 128
EPS = 1e-6
NEG = -1e30
LOG2E = 1.4426950408889634

BM_PROJ = 256
BM_Q = 512


def _rms_norm_heads(t, w, extra_scale):
    rows = t.shape[0]
    n = t.shape[1] // DH
    t3 = t.reshape(rows, n, DH)
    var = jnp.mean(t3 * t3, axis=-1, keepdims=True)
    t3 = t3 * (jax.lax.rsqrt(var + EPS) * extra_scale)
    return (t3 * w.reshape(1, 1, DH)).reshape(rows, n * DH)


def _qkv_proj_kernel(x_ref, wq_ref, wk_ref, wv_ref, qw_ref, kw_ref,
                     q_ref, k_ref, v_ref):
    xb = x_ref[...].astype(jnp.bfloat16)
    dims = (((1,), (1,)), ((), ()))
    q = jax.lax.dot_general(xb, wq_ref[...], dims,
                            preferred_element_type=jnp.float32)
    k = jax.lax.dot_general(xb, wk_ref[...], dims,
                            preferred_element_type=jnp.float32)
    v = jax.lax.dot_general(xb, wv_ref[...], dims,
                            preferred_element_type=jnp.float32)
    qn = _rms_norm_heads(q, qw_ref[...], DH ** -0.5 * LOG2E)
    kn = _rms_norm_heads(k, kw_ref[...], 1.0)
    q_ref[...] = qn.astype(jnp.bfloat16)
    k_ref[...] = kn.astype(jnp.bfloat16)
    v_ref[...] = v.astype(jnp.bfloat16)


def _attn_kernel(q_ref, k_ref, v_ref, o_ref):
    i = pl.program_id(1)
    dims_nt = (((1,), (1,)), ((), ()))
    dims_nn = (((1,), (0,)), ((), ()))
    q = q_ref[...]

    def chunk(j, carry):
        acc, l = carry
        kc = k_ref[pl.ds(j * BM_Q, BM_Q), :]
        s = jax.lax.dot_general(q, kc, dims_nt,
                                preferred_element_type=jnp.float32)
        p = jnp.exp2(s)
        l = l + jnp.sum(p, axis=-1, keepdims=True)
        vc = v_ref[pl.ds(j * BM_Q, BM_Q), :]
        acc = acc + jax.lax.dot_general(p.astype(jnp.bfloat16), vc, dims_nn,
                                        preferred_element_type=jnp.float32)
        return acc, l

    acc = jnp.zeros((BM_Q, DH), jnp.float32)
    l = jnp.zeros((BM_Q, 1), jnp.float32)
    acc, l = jax.lax.fori_loop(0, i, chunk, (acc, l))

    kc = k_ref[pl.ds(i * BM_Q, BM_Q), :]
    s = jax.lax.dot_general(q, kc, dims_nt,
                            preferred_element_type=jnp.float32)
    row = jax.lax.broadcasted_iota(jnp.int32, (BM_Q, BM_Q), 0)
    col = jax.lax.broadcasted_iota(jnp.int32, (BM_Q, BM_Q), 1)
    p = jnp.exp2(jnp.where(row >= col, s, NEG))
    l = l + jnp.sum(p, axis=-1, keepdims=True)
    vc = v_ref[pl.ds(i * BM_Q, BM_Q), :]
    acc = acc + jax.lax.dot_general(p.astype(jnp.bfloat16), vc, dims_nn,
                                    preferred_element_type=jnp.float32)
    o_ref[...] = (acc / l).astype(jnp.bfloat16)


def _out_proj_kernel(a_ref, wo_ref, o_ref):
    o_ref[...] = jax.lax.dot_general(
        a_ref[...], wo_ref[...], (((1,), (1,)), ((), ())),
        preferred_element_type=jnp.float32)


def kernel(x, Wq, Wk, Wv, Wo, q_norm_w, k_norm_w):
    b, s, d = x.shape
    x2 = x.reshape(s, d)
    wq = Wq.astype(jnp.bfloat16)
    wk = Wk.astype(jnp.bfloat16)
    wv = Wv.astype(jnp.bfloat16)
    wo = Wo.astype(jnp.bfloat16)
    qw = q_norm_w.reshape(1, DH)
    kw = k_norm_w.reshape(1, DH)

    n_row_blocks = s // BM_PROJ
    q, k, v = pl.pallas_call(
        _qkv_proj_kernel,
        grid=(n_row_blocks,),
        in_specs=[
            pl.BlockSpec((BM_PROJ, d), lambda i: (i, 0)),
            pl.BlockSpec((H * DH, d), lambda i: (0, 0)),
            pl.BlockSpec((KVH * DH, d), lambda i: (0, 0)),
            pl.BlockSpec((KVH * DH, d), lambda i: (0, 0)),
            pl.BlockSpec((1, DH), lambda i: (0, 0)),
            pl.BlockSpec((1, DH), lambda i: (0, 0)),
        ],
        out_specs=[
            pl.BlockSpec((BM_PROJ, H * DH), lambda i: (i, 0)),
            pl.BlockSpec((BM_PROJ, KVH * DH), lambda i: (i, 0)),
            pl.BlockSpec((BM_PROJ, KVH * DH), lambda i: (i, 0)),
        ],
        out_shape=[
            jax.ShapeDtypeStruct((s, H * DH), jnp.bfloat16),
            jax.ShapeDtypeStruct((s, KVH * DH), jnp.bfloat16),
            jax.ShapeDtypeStruct((s, KVH * DH), jnp.bfloat16),
        ],
    )(x2, wq, wk, wv, qw, kw)

    n_q_blocks = s // BM_Q
    groups = H // KVH
    attn = pl.pallas_call(
        _attn_kernel,
        grid=(H, n_q_blocks),
        in_specs=[
            pl.BlockSpec((BM_Q, DH), lambda h, i: (i, h)),
            pl.BlockSpec((s, DH), lambda h, i: (0, h // groups)),
            pl.BlockSpec((s, DH), lambda h, i: (0, h // groups)),
        ],
        out_specs=pl.BlockSpec((BM_Q, DH), lambda h, i: (i, h)),
        out_shape=jax.ShapeDtypeStruct((s, H * DH), jnp.bfloat16),
    )(q, k, v)

    out = pl.pallas_call(
        _out_proj_kernel,
        grid=(n_row_blocks,),
        in_specs=[
            pl.BlockSpec((BM_PROJ, H * DH), lambda i: (i, 0)),
            pl.BlockSpec((d, H * DH), lambda i: (0, 0)),
        ],
        out_specs=pl.BlockSpec((BM_PROJ, d), lambda i: (i, 0)),
        out_shape=jax.ShapeDtypeStruct((s, d), jnp.float32),
    )(attn, wo)

    return out.reshape(b, s, d)
